# Initial kernel scaffold; baseline (speedup 1.0000x reference)
import jax
import jax.numpy as jnp
from jax.experimental import pallas as pl


def kernel(x, edge_index, W, b):
    raise NotImplementedError("write your pallas kernel here")


# speedup vs baseline: 2.9014x; 2.9014x over previous
import functools

import jax
import jax.numpy as jnp
from jax import lax
from jax.experimental import pallas as pl
from jax.experimental.pallas import tpu as pltpu
from jax.experimental.pallas import tpu_sc as plsc

N = 10000
E = 320000
F = 128
NC = 2
NS = 16
NW = NC * NS
L = 16
B = 64
NB = 160
EPW = NB * B
AGGR = 10240
ZR = AGGR // NS


def _sc_agg_body(x_hbm, src_hbm, dst_hbm, zrow_hbm,
                 part_hbm, degp_hbm,
                 sidx, didx, buf, degloc, agg_sh, sem):
    c = lax.axis_index("c")
    s = lax.axis_index("s")
    wid = c * NS + s
    base = wid * EPW

    zeros16 = jnp.zeros((L,), jnp.float32)
    ones16 = jnp.ones((L,), jnp.float32)

    def zdeg(i, carry):
        degloc[pl.ds(i * L, L)] = zeros16
        return carry

    lax.fori_loop(0, AGGR // L, zdeg, 0)

    pltpu.sync_copy(zrow_hbm, buf)
    for k in range(ZR // B):
        pltpu.sync_copy(buf, agg_sh.at[pl.ds(s * ZR + k * B, B)])
    plsc.subcore_barrier()

    def body(j, carry):
        off = pl.multiple_of(base + j * B, B)
        pltpu.sync_copy(src_hbm.at[pl.ds(off, B)], sidx)
        pltpu.sync_copy(dst_hbm.at[pl.ds(off, B)], didx)
        pltpu.async_copy(x_hbm.at[sidx], buf, sem).wait()
        pltpu.sync_copy(buf, agg_sh.at[didx], add=True)
        for k in range(B // L):
            plsc.addupdate_scatter(degloc, [didx[pl.ds(k * L, L)]], ones16)
        return carry

    lax.fori_loop(0, NB, body, 0)
    plsc.subcore_barrier()

    for k in range(ZR // B):
        r = s * ZR + k * B
        o = pl.multiple_of(c * AGGR + r, B)
        pltpu.sync_copy(agg_sh.at[pl.ds(r, B)], buf)
        pltpu.sync_copy(buf, part_hbm.at[pl.ds(o, B)])
    pltpu.sync_copy(degloc, degp_hbm.at[pl.ds(wid * AGGR, AGGR)])


_sc_agg = functools.partial(
    pl.kernel,
    mesh=plsc.VectorSubcoreMesh(core_axis_name="c", subcore_axis_name="s"),
    compiler_params=pltpu.CompilerParams(needs_layout_passes=False),
    out_type=[
        jax.ShapeDtypeStruct((NC * AGGR, F), jnp.float32),
        jax.ShapeDtypeStruct((NW * AGGR,), jnp.float32),
    ],
    scratch_types=[
        pltpu.VMEM((B,), jnp.int32),
        pltpu.VMEM((B,), jnp.int32),
        pltpu.VMEM((B, F), jnp.float32),
        pltpu.VMEM((AGGR,), jnp.float32),
        pltpu.VMEM_SHARED((AGGR, F), jnp.float32),
        pltpu.SemaphoreType.DMA,
    ],
)(_sc_agg_body)


def _tc_dense_body(x_ref, p_ref, dp_ref, wx_ref, wa_ref, b_ref, o_ref):
    agg = p_ref[0] + p_ref[1]
    deg = jnp.sum(dp_ref[...], axis=1, keepdims=True)
    agg = agg / jnp.maximum(deg, 1.0)
    o_ref[...] = (
        jnp.dot(x_ref[...], wx_ref[...], preferred_element_type=jnp.float32)
        + jnp.dot(agg, wa_ref[...], preferred_element_type=jnp.float32)
        + b_ref[...]
    )


def _tc_dense(x, part, degt, wx, wa, b2):
    R = 1000
    grid = (N // R,)
    return pl.pallas_call(
        _tc_dense_body,
        grid=grid,
        in_specs=[
            pl.BlockSpec((R, F), lambda i: (i, 0)),
            pl.BlockSpec((NC, R, F), lambda i: (0, i, 0)),
            pl.BlockSpec((R, NW), lambda i: (i, 0)),
            pl.BlockSpec((F, F), lambda i: (0, 0)),
            pl.BlockSpec((F, F), lambda i: (0, 0)),
            pl.BlockSpec((1, F), lambda i: (0, 0)),
        ],
        out_specs=pl.BlockSpec((R, F), lambda i: (i, 0)),
        out_shape=jax.ShapeDtypeStruct((N, F), jnp.float32),
    )(x, part, degt, wx, wa, b2)


def kernel(x, edge_index, W, b):
    ei = edge_index.astype(jnp.int32)
    pad = NW * EPW - E
    src = jnp.concatenate([ei[0], jnp.zeros((pad,), jnp.int32)])
    dst = jnp.concatenate([ei[1], jnp.full((pad,), N, jnp.int32)])

    zrow = jnp.zeros((B, F), jnp.float32)

    part, degp = _sc_agg(x, src, dst, zrow)
    part = part.reshape(NC, AGGR, F)
    degt = degp.reshape(NW, AGGR).T

    wx = W[:, :F].T
    wa = W[:, F:].T
    return _tc_dense(x, part, degt, wx, wa, b.reshape(1, F))


# speedup vs baseline: 3.6619x; 1.2621x over previous
import functools

import jax
import jax.numpy as jnp
from jax import lax
from jax.experimental import pallas as pl
from jax.experimental.pallas import tpu as pltpu
from jax.experimental.pallas import tpu_sc as plsc

N = 10000
E = 320000
F = 128
NC = 2
NS = 16
NW = NC * NS
L = 16
B = 64
NB = 160
CH = 8
NCH = NB // CH
EPW = NB * B
AGGR = 10240
ZR = AGGR // NS


def _sc_agg_body(x_hbm, sd_hbm, zrow_hbm,
                 part_hbm, degp_hbm,
                 sd, buf0, buf1, degloc, agg_sh,
                 gsem0, gsem1, ssem0, ssem1):
    c = lax.axis_index("c")
    s = lax.axis_index("s")
    wid = c * NS + s
    base = wid * EPW

    zeros16 = jnp.zeros((L,), jnp.float32)
    ones16 = jnp.ones((L,), jnp.float32)

    def zdeg(i, carry):
        degloc[pl.ds(i * L, L)] = zeros16
        return carry

    lax.fori_loop(0, AGGR // L, zdeg, 0)

    pltpu.sync_copy(zrow_hbm, buf0)
    for k in range(ZR // B):
        pltpu.sync_copy(buf0, agg_sh.at[pl.ds(s * ZR + k * B, B)])
    plsc.subcore_barrier()

    bufs = (buf0, buf1)
    gsems = (gsem0, gsem1)
    ssems = (ssem0, ssem1)

    def chunk(ch, carry):
        pltpu.sync_copy(sd_hbm.at[wid, ch], sd)
        gd = [None] * CH
        sdn = [None] * CH
        gd[0] = pltpu.async_copy(x_hbm.at[sd.at[0]], bufs[0], gsems[0])
        for b in range(CH):
            p = b % 2
            gd[b].wait()
            sdn[b] = pltpu.async_copy(bufs[p], agg_sh.at[sd.at[CH + b]],
                                      ssems[p], add=True)
            if b + 1 < CH:
                if b >= 1:
                    sdn[b - 1].wait()
                gd[b + 1] = pltpu.async_copy(x_hbm.at[sd.at[b + 1]],
                                             bufs[1 - p], gsems[1 - p])
            for k in range(B // L):
                plsc.addupdate_scatter(degloc, [sd[CH + b, pl.ds(k * L, L)]],
                                       ones16)
        sdn[CH - 2].wait()
        sdn[CH - 1].wait()
        return carry

    lax.fori_loop(0, NCH, chunk, 0)
    plsc.subcore_barrier()

    for k in range(ZR // B):
        r = s * ZR + k * B
        o = pl.multiple_of(c * AGGR + r, B)
        pltpu.sync_copy(agg_sh.at[pl.ds(r, B)], buf0)
        pltpu.sync_copy(buf0, part_hbm.at[pl.ds(o, B)])
    pltpu.sync_copy(degloc, degp_hbm.at[pl.ds(wid * AGGR, AGGR)])


_sc_agg = functools.partial(
    pl.kernel,
    mesh=plsc.VectorSubcoreMesh(core_axis_name="c", subcore_axis_name="s"),
    compiler_params=pltpu.CompilerParams(needs_layout_passes=False),
    out_type=[
        jax.ShapeDtypeStruct((NC * AGGR, F), jnp.float32),
        jax.ShapeDtypeStruct((NW * AGGR,), jnp.float32),
    ],
    scratch_types=[
        pltpu.VMEM((2 * CH, B), jnp.int32),
        pltpu.VMEM((B, F), jnp.float32),
        pltpu.VMEM((B, F), jnp.float32),
        pltpu.VMEM((AGGR,), jnp.float32),
        pltpu.VMEM_SHARED((AGGR, F), jnp.float32),
        pltpu.SemaphoreType.DMA,
        pltpu.SemaphoreType.DMA,
        pltpu.SemaphoreType.DMA,
        pltpu.SemaphoreType.DMA,
    ],
)(_sc_agg_body)


def _tc_dense_body(x_ref, p_ref, dp_ref, wx_ref, wa_ref, b_ref, o_ref):
    agg = p_ref[0] + p_ref[1]
    deg = jnp.sum(dp_ref[...], axis=1, keepdims=True)
    agg = agg / jnp.maximum(deg, 1.0)
    o_ref[...] = (
        jnp.dot(x_ref[...], wx_ref[...], preferred_element_type=jnp.float32)
        + jnp.dot(agg, wa_ref[...], preferred_element_type=jnp.float32)
        + b_ref[...]
    )


def _tc_dense(x, part, degt, wx, wa, b2):
    R = 1000
    grid = (N // R,)
    return pl.pallas_call(
        _tc_dense_body,
        grid=grid,
        in_specs=[
            pl.BlockSpec((R, F), lambda i: (i, 0)),
            pl.BlockSpec((NC, R, F), lambda i: (0, i, 0)),
            pl.BlockSpec((R, NW), lambda i: (i, 0)),
            pl.BlockSpec((F, F), lambda i: (0, 0)),
            pl.BlockSpec((F, F), lambda i: (0, 0)),
            pl.BlockSpec((1, F), lambda i: (0, 0)),
        ],
        out_specs=pl.BlockSpec((R, F), lambda i: (i, 0)),
        out_shape=jax.ShapeDtypeStruct((N, F), jnp.float32),
    )(x, part, degt, wx, wa, b2)


def kernel(x, edge_index, W, b):
    ei = edge_index.astype(jnp.int32)
    pad = NW * EPW - E
    src = jnp.concatenate([ei[0], jnp.zeros((pad,), jnp.int32)])
    dst = jnp.concatenate([ei[1], jnp.full((pad,), N, jnp.int32)])
    sd = jnp.stack([src.reshape(NW, NCH, CH, B), dst.reshape(NW, NCH, CH, B)],
                   axis=2).reshape(NW, NCH, 2 * CH, B)

    zrow = jnp.zeros((B, F), jnp.float32)

    part, degp = _sc_agg(x, sd, zrow)
    part = part.reshape(NC, AGGR, F)
    degt = degp.reshape(NW, AGGR).T

    wx = W[:, :F].T
    wa = W[:, F:].T
    return _tc_dense(x, part, degt, wx, wa, b.reshape(1, F))


# speedup vs baseline: 3.9413x; 1.0763x over previous
import functools

import jax
import jax.numpy as jnp
from jax import lax
from jax.experimental import pallas as pl
from jax.experimental.pallas import tpu as pltpu
from jax.experimental.pallas import tpu_sc as plsc

N = 10000
E = 320000
F = 128
NC = 2
NS = 16
NW = NC * NS
L = 16
B = 64
NB = 160
CH = 8
NCH = NB // CH
EPW = NB * B
AGGR = 10240
ZR = AGGR // NS


def _sc_agg_body(x_hbm, sd_hbm, zrow_hbm,
                 part_hbm, degp_hbm,
                 sd, buf0, buf1, buf2, degloc, agg_sh,
                 gsem0, gsem1, gsem2, ssem0, ssem1, ssem2):
    c = lax.axis_index("c")
    s = lax.axis_index("s")
    wid = c * NS + s
    base = wid * EPW

    zeros16 = jnp.zeros((L,), jnp.float32)
    ones16 = jnp.ones((L,), jnp.float32)

    def zdeg(i, carry):
        degloc[pl.ds(i * L, L)] = zeros16
        return carry

    lax.fori_loop(0, AGGR // L, zdeg, 0)

    pltpu.sync_copy(zrow_hbm, buf0)
    for k in range(ZR // B):
        pltpu.sync_copy(buf0, agg_sh.at[pl.ds(s * ZR + k * B, B)])
    plsc.subcore_barrier()

    bufs = (buf0, buf1, buf2)
    gsems = (gsem0, gsem1, gsem2)
    ssems = (ssem0, ssem1, ssem2)

    def chunk(ch, carry):
        pltpu.sync_copy(sd_hbm.at[wid, ch], sd)
        gd = [None] * CH
        sdn = [None] * CH
        gd[0] = pltpu.async_copy(x_hbm.at[sd.at[0]], bufs[0], gsems[0])
        gd[1] = pltpu.async_copy(x_hbm.at[sd.at[1]], bufs[1], gsems[1])
        for b in range(CH):
            p = b % 3
            gd[b].wait()
            sdn[b] = pltpu.async_copy(bufs[p], agg_sh.at[sd.at[CH + b]],
                                      ssems[p], add=True)
            if b + 2 < CH:
                if b >= 1:
                    sdn[b - 1].wait()
                gd[b + 2] = pltpu.async_copy(x_hbm.at[sd.at[b + 2]],
                                             bufs[(b + 2) % 3],
                                             gsems[(b + 2) % 3])
            for k in range(B // L):
                plsc.addupdate_scatter(degloc, [sd[CH + b, pl.ds(k * L, L)]],
                                       ones16)
        sdn[CH - 3].wait()
        sdn[CH - 2].wait()
        sdn[CH - 1].wait()
        return carry

    lax.fori_loop(0, NCH, chunk, 0)
    plsc.subcore_barrier()

    for k in range(ZR // B):
        r = s * ZR + k * B
        o = pl.multiple_of(c * AGGR + r, B)
        pltpu.sync_copy(agg_sh.at[pl.ds(r, B)], buf0)
        pltpu.sync_copy(buf0, part_hbm.at[pl.ds(o, B)])
    pltpu.sync_copy(degloc, degp_hbm.at[pl.ds(wid * AGGR, AGGR)])


_sc_agg = functools.partial(
    pl.kernel,
    mesh=plsc.VectorSubcoreMesh(core_axis_name="c", subcore_axis_name="s"),
    compiler_params=pltpu.CompilerParams(needs_layout_passes=False),
    out_type=[
        jax.ShapeDtypeStruct((NC * AGGR, F), jnp.float32),
        jax.ShapeDtypeStruct((NW * AGGR,), jnp.float32),
    ],
    scratch_types=[
        pltpu.VMEM((2 * CH, B), jnp.int32),
        pltpu.VMEM((B, F), jnp.float32),
        pltpu.VMEM((B, F), jnp.float32),
        pltpu.VMEM((B, F), jnp.float32),
        pltpu.VMEM((AGGR,), jnp.float32),
        pltpu.VMEM_SHARED((AGGR, F), jnp.float32),
        pltpu.SemaphoreType.DMA,
        pltpu.SemaphoreType.DMA,
        pltpu.SemaphoreType.DMA,
        pltpu.SemaphoreType.DMA,
        pltpu.SemaphoreType.DMA,
        pltpu.SemaphoreType.DMA,
    ],
)(_sc_agg_body)


def _tc_dense_body(x_ref, p_ref, dp_ref, wx_ref, wa_ref, b_ref, o_ref):
    agg = p_ref[0] + p_ref[1]
    deg = jnp.sum(dp_ref[...], axis=1, keepdims=True)
    agg = agg / jnp.maximum(deg, 1.0)
    o_ref[...] = (
        jnp.dot(x_ref[...], wx_ref[...], preferred_element_type=jnp.float32)
        + jnp.dot(agg, wa_ref[...], preferred_element_type=jnp.float32)
        + b_ref[...]
    )


def _tc_dense(x, part, degt, wx, wa, b2):
    R = 1000
    grid = (N // R,)
    return pl.pallas_call(
        _tc_dense_body,
        grid=grid,
        in_specs=[
            pl.BlockSpec((R, F), lambda i: (i, 0)),
            pl.BlockSpec((NC, R, F), lambda i: (0, i, 0)),
            pl.BlockSpec((R, NW), lambda i: (i, 0)),
            pl.BlockSpec((F, F), lambda i: (0, 0)),
            pl.BlockSpec((F, F), lambda i: (0, 0)),
            pl.BlockSpec((1, F), lambda i: (0, 0)),
        ],
        out_specs=pl.BlockSpec((R, F), lambda i: (i, 0)),
        out_shape=jax.ShapeDtypeStruct((N, F), jnp.float32),
    )(x, part, degt, wx, wa, b2)


def kernel(x, edge_index, W, b):
    ei = edge_index.astype(jnp.int32)
    pad = NW * EPW - E
    src = jnp.concatenate([ei[0], jnp.zeros((pad,), jnp.int32)])
    dst = jnp.concatenate([ei[1], jnp.full((pad,), N, jnp.int32)])
    sd = jnp.stack([src.reshape(NW, NCH, CH, B), dst.reshape(NW, NCH, CH, B)],
                   axis=2).reshape(NW, NCH, 2 * CH, B)

    zrow = jnp.zeros((B, F), jnp.float32)

    part, degp = _sc_agg(x, sd, zrow)
    part = part.reshape(NC, AGGR, F)
    degt = degp.reshape(NW, AGGR).T

    wx = W[:, :F].T
    wa = W[:, F:].T
    return _tc_dense(x, part, degt, wx, wa, b.reshape(1, F))


# speedup vs baseline: 3.9616x; 1.0052x over previous
import functools

import jax
import jax.numpy as jnp
from jax import lax
from jax.experimental import pallas as pl
from jax.experimental.pallas import tpu as pltpu
from jax.experimental.pallas import tpu_sc as plsc

N = 10000
E = 320000
F = 128
NC = 2
NS = 16
NW = NC * NS
L = 16
B = 64
NB = 160
CH = 8
NCH = NB // CH
EPW = NB * B
AGGR = 10240
ZR = AGGR // NS


def _sc_agg_body(x_hbm, sd_hbm, zrow_hbm,
                 part_hbm, degp_hbm,
                 sd, buf0, buf1, buf2, degloc, agg_sh,
                 gsem0, gsem1, gsem2, ssem0, ssem1, ssem2):
    c = lax.axis_index("c")
    s = lax.axis_index("s")
    wid = c * NS + s
    base = wid * EPW

    zeros16 = jnp.zeros((L,), jnp.float32)
    ones16 = jnp.ones((L,), jnp.float32)

    def zdeg(i, carry):
        degloc[pl.ds(i * L, L)] = zeros16
        return carry

    lax.fori_loop(0, AGGR // L, zdeg, 0)

    pltpu.sync_copy(zrow_hbm, buf0)
    zds = [pltpu.async_copy(buf0, agg_sh.at[pl.ds(s * ZR + k * B, B)], gsem0)
           for k in range(ZR // B)]
    for d in zds:
        d.wait()
    plsc.subcore_barrier()

    bufs = (buf0, buf1, buf2)
    gsems = (gsem0, gsem1, gsem2)
    ssems = (ssem0, ssem1, ssem2)

    def chunk(ch, carry):
        pltpu.sync_copy(sd_hbm.at[wid, ch], sd)
        gd = [None] * CH
        sdn = [None] * CH
        gd[0] = pltpu.async_copy(x_hbm.at[sd.at[0]], bufs[0], gsems[0])
        gd[1] = pltpu.async_copy(x_hbm.at[sd.at[1]], bufs[1], gsems[1])
        for b in range(CH):
            p = b % 3
            gd[b].wait()
            sdn[b] = pltpu.async_copy(bufs[p], agg_sh.at[sd.at[CH + b]],
                                      ssems[p], add=True)
            if b + 2 < CH:
                if b >= 1:
                    sdn[b - 1].wait()
                gd[b + 2] = pltpu.async_copy(x_hbm.at[sd.at[b + 2]],
                                             bufs[(b + 2) % 3],
                                             gsems[(b + 2) % 3])
            for k in range(B // L):
                plsc.addupdate_scatter(degloc, [sd[CH + b, pl.ds(k * L, L)]],
                                       ones16)
        sdn[CH - 3].wait()
        sdn[CH - 2].wait()
        sdn[CH - 1].wait()
        return carry

    lax.fori_loop(0, NCH, chunk, 0)
    plsc.subcore_barrier()

    NK = ZR // B
    rds = [None] * NK
    wrs = [None] * NK
    dd = pltpu.async_copy(degloc, degp_hbm.at[pl.ds(wid * AGGR, AGGR)], gsem2)
    rds[0] = pltpu.async_copy(agg_sh.at[pl.ds(s * ZR, B)], buf0, gsem0)
    for k in range(NK):
        p = k % 2
        r = s * ZR + k * B
        o = pl.multiple_of(c * AGGR + r, B)
        rds[k].wait()
        wrs[k] = pltpu.async_copy(bufs[p], part_hbm.at[pl.ds(o, B)], ssems[p])
        if k + 1 < NK:
            if k >= 1:
                wrs[k - 1].wait()
            r2 = s * ZR + (k + 1) * B
            rds[k + 1] = pltpu.async_copy(agg_sh.at[pl.ds(r2, B)],
                                          bufs[1 - p], gsems[1 - p])
    wrs[NK - 2].wait()
    wrs[NK - 1].wait()
    dd.wait()


_sc_agg = functools.partial(
    pl.kernel,
    mesh=plsc.VectorSubcoreMesh(core_axis_name="c", subcore_axis_name="s"),
    compiler_params=pltpu.CompilerParams(needs_layout_passes=False),
    out_type=[
        jax.ShapeDtypeStruct((NC * AGGR, F), jnp.float32),
        jax.ShapeDtypeStruct((NW * AGGR,), jnp.float32),
    ],
    scratch_types=[
        pltpu.VMEM((2 * CH, B), jnp.int32),
        pltpu.VMEM((B, F), jnp.float32),
        pltpu.VMEM((B, F), jnp.float32),
        pltpu.VMEM((B, F), jnp.float32),
        pltpu.VMEM((AGGR,), jnp.float32),
        pltpu.VMEM_SHARED((AGGR, F), jnp.float32),
        pltpu.SemaphoreType.DMA,
        pltpu.SemaphoreType.DMA,
        pltpu.SemaphoreType.DMA,
        pltpu.SemaphoreType.DMA,
        pltpu.SemaphoreType.DMA,
        pltpu.SemaphoreType.DMA,
    ],
)(_sc_agg_body)


def _tc_dense_body(x_ref, p_ref, dp_ref, wx_ref, wa_ref, b_ref, o_ref):
    agg = p_ref[0] + p_ref[1]
    deg = jnp.sum(dp_ref[...], axis=1, keepdims=True)
    agg = agg / jnp.maximum(deg, 1.0)
    o_ref[...] = (
        jnp.dot(x_ref[...], wx_ref[...], preferred_element_type=jnp.float32)
        + jnp.dot(agg, wa_ref[...], preferred_element_type=jnp.float32)
        + b_ref[...]
    )


def _tc_dense(x, part, degt, wx, wa, b2):
    R = 1000
    grid = (N // R,)
    return pl.pallas_call(
        _tc_dense_body,
        grid=grid,
        in_specs=[
            pl.BlockSpec((R, F), lambda i: (i, 0)),
            pl.BlockSpec((NC, R, F), lambda i: (0, i, 0)),
            pl.BlockSpec((R, NW), lambda i: (i, 0)),
            pl.BlockSpec((F, F), lambda i: (0, 0)),
            pl.BlockSpec((F, F), lambda i: (0, 0)),
            pl.BlockSpec((1, F), lambda i: (0, 0)),
        ],
        out_specs=pl.BlockSpec((R, F), lambda i: (i, 0)),
        out_shape=jax.ShapeDtypeStruct((N, F), jnp.float32),
    )(x, part, degt, wx, wa, b2)


def kernel(x, edge_index, W, b):
    ei = edge_index.astype(jnp.int32)
    pad = NW * EPW - E
    src = jnp.concatenate([ei[0], jnp.zeros((pad,), jnp.int32)])
    dst = jnp.concatenate([ei[1], jnp.full((pad,), N, jnp.int32)])
    sd = jnp.stack([src.reshape(NW, NCH, CH, B), dst.reshape(NW, NCH, CH, B)],
                   axis=2).reshape(NW, NCH, 2 * CH, B)

    zrow = jnp.zeros((B, F), jnp.float32)

    part, degp = _sc_agg(x, sd, zrow)
    part = part.reshape(NC, AGGR, F)
    degt = degp.reshape(NW, AGGR).T

    wx = W[:, :F].T
    wa = W[:, F:].T
    return _tc_dense(x, part, degt, wx, wa, b.reshape(1, F))
